# baseline (speedup 1.0000x reference)
import functools

import jax
import jax.numpy as jnp
from jax.experimental import pallas as pl
from jax.experimental.pallas import tpu as pltpu

_NSPLIT = 5
_SUB = 200


def _fused_kernel(*refs):
    x_refs = refs[:_NSPLIT]
    hg_refs = refs[_NSPLIT:2 * _NSPLIT]
    w1_ref, b1_ref, w2_ref, b2_ref, out_ref, acc_ref, y_ref = refs[2 * _NSPLIT:]
    s = pl.program_id(0)
    n = pl.program_id(1)
    nsteps = pl.num_programs(1)
    C = w1_ref.shape[0]

    @pl.when(s == 0)
    def _v2e():
        part = None
        for k in range(_NSPLIT):
            h16 = hg_refs[k][...].astype(jnp.bfloat16)
            x1 = jnp.dot(x_refs[k][...], w1_ref[...],
                         preferred_element_type=jnp.float32) + b1_ref[...]
            x1aug = jnp.concatenate(
                [x1, jnp.ones((_SUB, 8), jnp.float32)],
                axis=1).astype(jnp.bfloat16)
            p = jax.lax.dot_general(
                x1aug, h16, (((0,), (0,)), ((), ())),
                preferred_element_type=jnp.float32)
            part = p if part is None else part + p

        @pl.when(n == 0)
        def _init():
            acc_ref[...] = part

        @pl.when(n > 0)
        def _acc():
            acc_ref[...] += part

        @pl.when(n == nsteps - 1)
        def _finish():
            de = acc_ref[C:C + 1, :]
            y = jnp.maximum(acc_ref[:C, :] * (1.0 / de), 0.0)
            y2 = jax.lax.dot_general(
                w2_ref[...].astype(jnp.bfloat16), y.astype(jnp.bfloat16),
                (((0,), (0,)), ((), ())),
                preferred_element_type=jnp.float32) + b2_ref[...]
            y_ref[...] = jnp.transpose(y2).astype(jnp.bfloat16)

    @pl.when(s == 1)
    def _e2v():
        for k in range(_NSPLIT):
            h = hg_refs[k][...]
            xv = jnp.dot(h.astype(jnp.bfloat16), y_ref[...],
                         preferred_element_type=jnp.float32)
            dv = jnp.sum(h, axis=1, keepdims=True)
            scale = jnp.where(dv > 0.0, 1.0 / dv, 0.0)
            out_ref[k * _SUB:(k + 1) * _SUB, :] = jnp.maximum(xv * scale, 0.0)


@jax.jit
def kernel(X, hg, W_v2e, b_v2e, W_e2v, b_e2v):
    N, C = X.shape
    M = hg.shape[1]
    blk = _NSPLIT * _SUB
    assert N % blk == 0

    b1 = b_v2e.reshape(1, C)
    b2 = b_e2v.reshape(C, 1)

    def xmap(k):
        return lambda s, n: (_NSPLIT * n + k, 0)

    x_specs = [pl.BlockSpec((_SUB, C), xmap(k)) for k in range(_NSPLIT)]
    hg_specs = [pl.BlockSpec((_SUB, M), xmap(k)) for k in range(_NSPLIT)]

    xv = pl.pallas_call(
        _fused_kernel,
        grid=(2, N // blk),
        in_specs=x_specs + hg_specs + [
            pl.BlockSpec((C, C), lambda s, n: (0, 0)),
            pl.BlockSpec((1, C), lambda s, n: (0, 0)),
            pl.BlockSpec((C, C), lambda s, n: (0, 0)),
            pl.BlockSpec((C, 1), lambda s, n: (0, 0)),
        ],
        out_specs=pl.BlockSpec((blk, C), lambda s, n: (n, 0)),
        out_shape=jax.ShapeDtypeStruct((N, C), jnp.float32),
        scratch_shapes=[
            pltpu.VMEM((C + 8, M), jnp.float32),
            pltpu.VMEM((M, C), jnp.bfloat16),
        ],
        compiler_params=pltpu.CompilerParams(
            dimension_semantics=("arbitrary", "arbitrary")),
    )(*([X] * _NSPLIT + [hg] * _NSPLIT + [W_v2e, b1, W_e2v, b2]))

    return xv
